# baseline (speedup 1.0000x reference)
import functools

import jax
import jax.numpy as jnp
from jax.experimental import pallas as pl


def _pool_proj_kernel(x_ref, w_ref, y_ref):
    pooled = jnp.max(x_ref[...], axis=2)
    y_ref[...] = jnp.dot(pooled, w_ref[...], preferred_element_type=jnp.float32)


def _pool_proj(x, w, block_n):
    n, c, hw = x.shape
    return pl.pallas_call(
        _pool_proj_kernel,
        grid=(n // block_n,),
        in_specs=[
            pl.BlockSpec((block_n, c, hw), lambda i: (i, 0, 0)),
            pl.BlockSpec((c, c), lambda i: (0, 0)),
        ],
        out_specs=pl.BlockSpec((block_n, c), lambda i: (i, 0)),
        out_shape=jax.ShapeDtypeStruct((n, c), jnp.float32),
    )(x, w)


def _graph_kernel(rois_ref, roisT_ref, tse_ref, tseT_ref, y_ref, b_ref,
                  out_ref, *, n, t, c, block_r):
    nb = n // block_r
    f32 = jnp.float32

    idx_l = jax.lax.broadcasted_iota(jnp.int32, (1, n), 1)
    cx_l = (roisT_ref[0:1, :] + roisT_ref[2:3, :]) * 0.5
    cy_l = (roisT_ref[1:2, :] + roisT_ref[3:4, :]) * 0.5
    starts_c = tse_ref[:, 0:1]
    ends_c = tse_ref[:, 1:2]
    m_cols = (idx_l >= starts_c) & (idx_l < ends_c)
    t_col = jax.lax.broadcasted_iota(jnp.int32, (t, 1), 0)
    seg_l = jnp.sum(jnp.where(m_cols, t_col, 0), axis=0, keepdims=True)
    in_l = jnp.sum(m_cols.astype(jnp.int32), axis=0, keepdims=True) > 0

    starts_l = tseT_ref[0:1, :]
    ends_l = tseT_ref[1:2, :]
    t_lane = jax.lax.broadcasted_iota(jnp.int32, (1, t), 1)

    def a_block(i):
        off = i * block_r
        idx_c = jax.lax.broadcasted_iota(jnp.int32, (block_r, 1), 0) + off
        cx_c = (rois_ref[pl.ds(off, block_r), 0:1]
                + rois_ref[pl.ds(off, block_r), 2:3]) * 0.5
        cy_c = (rois_ref[pl.ds(off, block_r), 1:2]
                + rois_ref[pl.ds(off, block_r), 3:4]) * 0.5
        m_rows = (idx_c >= starts_l) & (idx_c < ends_l)
        seg_c = jnp.sum(jnp.where(m_rows, t_lane, 0), axis=1, keepdims=True)
        in_c = jnp.sum(m_rows.astype(jnp.int32), axis=1, keepdims=True) > 0
        d2 = (cx_c - cx_l) ** 2 + (cy_c - cy_l) ** 2
        same = (seg_c == seg_l) & in_c & in_l
        a = jnp.where(same, jnp.exp(-d2), 0.0)
        a = a + (idx_c == idx_l).astype(f32)
        return a, idx_c

    def deg_body(i, colsum):
        a, _ = a_block(i)
        return colsum + jnp.sum(a, axis=0, keepdims=True)

    deg = jax.lax.fori_loop(0, nb, deg_body, jnp.zeros((1, n), f32))
    dinv_l = jnp.where(deg > 0, jax.lax.rsqrt(deg), 0.0)

    y = y_ref[...]
    bias = b_ref[...]

    def row_body(i, carry):
        acc, cnt = carry
        a, idx_c = a_block(i)
        rdeg = jnp.sum(a, axis=1, keepdims=True)
        dinv_c = jnp.where(rdeg > 0, jax.lax.rsqrt(rdeg), 0.0)
        an = a * dinv_l
        h = jnp.dot(an, y, preferred_element_type=f32)
        h = jnp.maximum(h * dinv_c + bias, 0.0)
        off = i * block_r
        idx_blk = jax.lax.broadcasted_iota(jnp.int32, (1, block_r), 1) + off
        mf = ((idx_blk >= starts_c) & (idx_blk < ends_c)).astype(f32)
        acc = acc + jnp.dot(mf, h, preferred_element_type=f32)
        cnt = cnt + jnp.sum(mf, axis=1, keepdims=True)
        return acc, cnt

    acc, cnt = jax.lax.fori_loop(
        0, nb, row_body,
        (jnp.zeros((t, c), f32), jnp.zeros((t, 1), f32)))
    out_ref[...] = acc / jnp.maximum(cnt, 1.0)


def _graph(rois, rois_t, tse, tse_t, y, b2d, block_r):
    n, c = y.shape
    t = tse.shape[0]
    return pl.pallas_call(
        functools.partial(_graph_kernel, n=n, t=t, c=c, block_r=block_r),
        in_specs=[
            pl.BlockSpec(rois.shape, lambda: (0, 0)),
            pl.BlockSpec(rois_t.shape, lambda: (0, 0)),
            pl.BlockSpec(tse.shape, lambda: (0, 0)),
            pl.BlockSpec(tse_t.shape, lambda: (0, 0)),
            pl.BlockSpec(y.shape, lambda: (0, 0)),
            pl.BlockSpec(b2d.shape, lambda: (0, 0)),
        ],
        out_specs=pl.BlockSpec((t, c), lambda: (0, 0)),
        out_shape=jax.ShapeDtypeStruct((t, c), jnp.float32),
    )(rois, rois_t, tse, tse_t, y, b2d)


@jax.jit
def kernel(rois, rois_features, temporal_start_end, W, b):
    n, c, hh, ww = rois_features.shape
    x = rois_features.reshape(n, c, hh * ww)
    y = _pool_proj(x, W, block_n=32)

    tse = temporal_start_end.astype(jnp.int32)
    out = _graph(rois, rois.T, tse, tse.T, y,
                 b.reshape(1, c).astype(jnp.float32), block_r=256)
    return out


# speedup vs baseline: 4.4008x; 4.4008x over previous
import functools

import jax
import jax.numpy as jnp
from jax.experimental import pallas as pl


def _pool_proj_kernel(x_ref, w_ref, y_ref):
    pooled = jnp.max(x_ref[...], axis=0)
    y_ref[...] = jnp.dot(pooled, w_ref[...], preferred_element_type=jnp.float32)


def _pool_proj(x, w, block_n):
    hw, n, c = x.shape
    return pl.pallas_call(
        _pool_proj_kernel,
        grid=(n // block_n,),
        in_specs=[
            pl.BlockSpec((hw, block_n, c), lambda i: (0, i, 0)),
            pl.BlockSpec((c, c), lambda i: (0, 0)),
        ],
        out_specs=pl.BlockSpec((block_n, c), lambda i: (i, 0)),
        out_shape=jax.ShapeDtypeStruct((n, c), jnp.float32),
    )(x, w)


def _graph_kernel(rois_ref, roisT_ref, tse_ref, tseT_ref, y_ref, b_ref,
                  out_ref, *, n, t, c, block_r):
    nb = n // block_r
    f32 = jnp.float32

    idx_l = jax.lax.broadcasted_iota(jnp.int32, (1, n), 1)
    cx_l = (roisT_ref[0:1, :] + roisT_ref[2:3, :]) * 0.5
    cy_l = (roisT_ref[1:2, :] + roisT_ref[3:4, :]) * 0.5
    starts_c = tse_ref[:, 0:1]
    ends_c = tse_ref[:, 1:2]
    m_cols = (idx_l >= starts_c) & (idx_l < ends_c)
    t_col = jax.lax.broadcasted_iota(jnp.int32, (t, 1), 0)
    seg_l = jnp.sum(jnp.where(m_cols, t_col, 0), axis=0, keepdims=True)
    in_l = jnp.sum(m_cols.astype(jnp.int32), axis=0, keepdims=True) > 0

    starts_l = tseT_ref[0:1, :]
    ends_l = tseT_ref[1:2, :]
    t_lane = jax.lax.broadcasted_iota(jnp.int32, (1, t), 1)

    def a_block(i):
        off = i * block_r
        idx_c = jax.lax.broadcasted_iota(jnp.int32, (block_r, 1), 0) + off
        cx_c = (rois_ref[pl.ds(off, block_r), 0:1]
                + rois_ref[pl.ds(off, block_r), 2:3]) * 0.5
        cy_c = (rois_ref[pl.ds(off, block_r), 1:2]
                + rois_ref[pl.ds(off, block_r), 3:4]) * 0.5
        m_rows = (idx_c >= starts_l) & (idx_c < ends_l)
        seg_c = jnp.sum(jnp.where(m_rows, t_lane, 0), axis=1, keepdims=True)
        in_c = jnp.sum(m_rows.astype(jnp.int32), axis=1, keepdims=True) > 0
        d2 = (cx_c - cx_l) ** 2 + (cy_c - cy_l) ** 2
        same = (seg_c == seg_l) & in_c & in_l
        a = jnp.where(same, jnp.exp(-d2), 0.0)
        a = a + (idx_c == idx_l).astype(f32)
        return a, idx_c

    def deg_body(i, colsum):
        a, _ = a_block(i)
        return colsum + jnp.sum(a, axis=0, keepdims=True)

    deg = jax.lax.fori_loop(0, nb, deg_body, jnp.zeros((1, n), f32))
    dinv_l = jnp.where(deg > 0, jax.lax.rsqrt(deg), 0.0)

    y = y_ref[...]
    bias = b_ref[...]

    def row_body(i, carry):
        acc, cnt = carry
        a, idx_c = a_block(i)
        rdeg = jnp.sum(a, axis=1, keepdims=True)
        dinv_c = jnp.where(rdeg > 0, jax.lax.rsqrt(rdeg), 0.0)
        an = a * dinv_l
        h = jnp.dot(an, y, preferred_element_type=f32)
        h = jnp.maximum(h * dinv_c + bias, 0.0)
        off = i * block_r
        idx_blk = jax.lax.broadcasted_iota(jnp.int32, (1, block_r), 1) + off
        mf = ((idx_blk >= starts_c) & (idx_blk < ends_c)).astype(f32)
        acc = acc + jnp.dot(mf, h, preferred_element_type=f32)
        cnt = cnt + jnp.sum(mf, axis=1, keepdims=True)
        return acc, cnt

    acc, cnt = jax.lax.fori_loop(
        0, nb, row_body,
        (jnp.zeros((t, c), f32), jnp.zeros((t, 1), f32)))
    out_ref[...] = acc / jnp.maximum(cnt, 1.0)


def _graph(rois, rois_t, tse, tse_t, y, b2d, block_r):
    n, c = y.shape
    t = tse.shape[0]
    return pl.pallas_call(
        functools.partial(_graph_kernel, n=n, t=t, c=c, block_r=block_r),
        in_specs=[
            pl.BlockSpec(rois.shape, lambda: (0, 0)),
            pl.BlockSpec(rois_t.shape, lambda: (0, 0)),
            pl.BlockSpec(tse.shape, lambda: (0, 0)),
            pl.BlockSpec(tse_t.shape, lambda: (0, 0)),
            pl.BlockSpec(y.shape, lambda: (0, 0)),
            pl.BlockSpec(b2d.shape, lambda: (0, 0)),
        ],
        out_specs=pl.BlockSpec((t, c), lambda: (0, 0)),
        out_shape=jax.ShapeDtypeStruct((t, c), jnp.float32),
    )(rois, rois_t, tse, tse_t, y, b2d)


@jax.jit
def kernel(rois, rois_features, temporal_start_end, W, b):
    n, c, hh, ww = rois_features.shape
    x = rois_features.transpose(2, 3, 0, 1).reshape(hh * ww, n, c)
    y = _pool_proj(x, W, block_n=64)

    tse = temporal_start_end.astype(jnp.int32)
    out = _graph(rois, rois.T, tse, tse.T, y,
                 b.reshape(1, c).astype(jnp.float32), block_r=256)
    return out


# speedup vs baseline: 4.7379x; 1.0766x over previous
import functools

import jax
import jax.numpy as jnp
from jax.experimental import pallas as pl
from jax.experimental.pallas import tpu as pltpu


def _fused_kernel(x_ref, w_ref, rois_ref, cxr_ref, cyr_ref, tse_ref,
                  tseT_ref, b_ref, out_ref, hacc, rowsum, seg_scr, cx_scr,
                  cy_scr, *, n, t, c, block_n):
    f32 = jnp.float32
    s = pl.program_id(0)
    nb = pl.num_programs(0)

    pooled = jnp.max(x_ref[...], axis=0)
    ys = jnp.dot(pooled, w_ref[...], preferred_element_type=f32)

    @pl.when(s == 0)
    def _():
        idx_c = jax.lax.broadcasted_iota(jnp.int32, (n, 1), 0)
        cx_scr[...] = (rois_ref[:, 0:1] + rois_ref[:, 2:3]) * 0.5
        cy_scr[...] = (rois_ref[:, 1:2] + rois_ref[:, 3:4]) * 0.5
        starts_l = tseT_ref[0:1, :]
        ends_l = tseT_ref[1:2, :]
        m = (idx_c >= starts_l) & (idx_c < ends_l)
        t_l = jax.lax.broadcasted_iota(jnp.int32, (1, t), 1)
        sid = jnp.sum(jnp.where(m, t_l, 0), axis=1, keepdims=True)
        assigned = jnp.sum(m.astype(jnp.int32), axis=1, keepdims=True) > 0
        seg_scr[...] = jnp.where(assigned, sid, -1)

    off = s * block_n
    idx_c = jax.lax.broadcasted_iota(jnp.int32, (n, 1), 0)
    idx_r = jax.lax.broadcasted_iota(jnp.int32, (1, block_n), 1) + off
    cx_r = cxr_ref[0]
    cy_r = cyr_ref[0]
    starts_c = tse_ref[:, 0:1]
    ends_c = tse_ref[:, 1:2]
    mr = (idx_r >= starts_c) & (idx_r < ends_c)
    t_c = jax.lax.broadcasted_iota(jnp.int32, (t, 1), 0)
    seg_r = jnp.sum(jnp.where(mr, t_c, 0), axis=0, keepdims=True)
    in_r = jnp.sum(mr.astype(jnp.int32), axis=0, keepdims=True) > 0

    d2 = (cx_scr[...] - cx_r) ** 2 + (cy_scr[...] - cy_r) ** 2
    same = (seg_scr[...] == seg_r) & (seg_scr[...] >= 0) & in_r
    a = jnp.where(same, jnp.exp(-d2), 0.0)
    a = a + (idx_c == idx_r).astype(f32)

    deg_b = jnp.sum(a, axis=0, keepdims=True)
    rsum = jnp.sum(a, axis=1, keepdims=True)
    contrib = jnp.dot(a * jax.lax.rsqrt(deg_b), ys,
                      preferred_element_type=f32)

    @pl.when(s == 0)
    def _():
        hacc[...] = contrib
        rowsum[...] = rsum

    @pl.when(s != 0)
    def _():
        hacc[...] += contrib
        rowsum[...] += rsum

    @pl.when(s == nb - 1)
    def _():
        h = jnp.maximum(hacc[...] * jax.lax.rsqrt(rowsum[...]) + b_ref[...],
                        0.0)
        idx_l = jax.lax.broadcasted_iota(jnp.int32, (1, n), 1)
        mf = ((idx_l >= starts_c) & (idx_l < ends_c)).astype(f32)
        acc = jnp.dot(mf, h, preferred_element_type=f32)
        cnt = jnp.sum(mf, axis=1, keepdims=True)
        out_ref[...] = acc / jnp.maximum(cnt, 1.0)


@jax.jit
def kernel(rois, rois_features, temporal_start_end, W, b):
    n, c, hh, ww = rois_features.shape
    hw = hh * ww
    t = temporal_start_end.shape[0]
    block_n = 64
    x = rois_features.transpose(2, 3, 0, 1).reshape(hw, n, c)
    tse = temporal_start_end.astype(jnp.int32)
    nb = n // block_n
    cxr = ((rois[:, 0] + rois[:, 2]) * 0.5).reshape(nb, 1, block_n)
    cyr = ((rois[:, 1] + rois[:, 3]) * 0.5).reshape(nb, 1, block_n)

    return pl.pallas_call(
        functools.partial(_fused_kernel, n=n, t=t, c=c, block_n=block_n),
        grid=(n // block_n,),
        in_specs=[
            pl.BlockSpec((hw, block_n, c), lambda i: (0, i, 0)),
            pl.BlockSpec((c, c), lambda i: (0, 0)),
            pl.BlockSpec((n, 4), lambda i: (0, 0)),
            pl.BlockSpec((1, 1, block_n), lambda i: (i, 0, 0)),
            pl.BlockSpec((1, 1, block_n), lambda i: (i, 0, 0)),
            pl.BlockSpec((t, 2), lambda i: (0, 0)),
            pl.BlockSpec((2, t), lambda i: (0, 0)),
            pl.BlockSpec((1, c), lambda i: (0, 0)),
        ],
        out_specs=pl.BlockSpec((t, c), lambda i: (0, 0)),
        out_shape=jax.ShapeDtypeStruct((t, c), jnp.float32),
        scratch_shapes=[
            pltpu.VMEM((n, c), jnp.float32),
            pltpu.VMEM((n, 1), jnp.float32),
            pltpu.VMEM((n, 1), jnp.int32),
            pltpu.VMEM((n, 1), jnp.float32),
            pltpu.VMEM((n, 1), jnp.float32),
        ],
    )(x, W, rois, cxr, cyr, tse, tse.T, b.reshape(1, c).astype(jnp.float32))


# speedup vs baseline: 5.1160x; 1.0798x over previous
import functools

import jax
import jax.numpy as jnp
from jax.experimental import pallas as pl
from jax.experimental.pallas import tpu as pltpu


def _fused_kernel(x_ref, w_ref, rois_ref, cxr_ref, cyr_ref, tse_ref,
                  tseT_ref, b_ref, out_ref, hacc, col_scr,
                  *, n, t, c, block_n):
    f32 = jnp.float32
    s = pl.program_id(0)
    nb = pl.num_programs(0)

    pooled = jnp.max(x_ref[...], axis=0)
    ys = jnp.dot(pooled, w_ref[...], preferred_element_type=f32)

    @pl.when(s == 0)
    def _():
        idx_c = jax.lax.broadcasted_iota(jnp.int32, (n, 1), 0)
        col_scr[:, 0:1] = (rois_ref[:, 0:1] + rois_ref[:, 2:3]) * 0.5
        col_scr[:, 1:2] = (rois_ref[:, 1:2] + rois_ref[:, 3:4]) * 0.5
        starts_l = tseT_ref[0:1, :]
        ends_l = tseT_ref[1:2, :]
        m = (idx_c >= starts_l) & (idx_c < ends_l)
        t_l = jax.lax.broadcasted_iota(jnp.int32, (1, t), 1)
        sid = jnp.sum(jnp.where(m, t_l, 0), axis=1, keepdims=True)
        assigned = jnp.sum(m.astype(jnp.int32), axis=1, keepdims=True) > 0
        col_scr[:, 2:3] = jnp.where(assigned, sid, -1).astype(f32)

    off = s * block_n
    idx_c = jax.lax.broadcasted_iota(jnp.int32, (n, 1), 0)
    idx_r = jax.lax.broadcasted_iota(jnp.int32, (1, block_n), 1) + off
    cx_r = cxr_ref[0]
    cy_r = cyr_ref[0]
    starts_c = tse_ref[:, 0:1]
    ends_c = tse_ref[:, 1:2]
    mr = (idx_r >= starts_c) & (idx_r < ends_c)
    t_c = jax.lax.broadcasted_iota(jnp.int32, (t, 1), 0)
    seg_r = jnp.sum(jnp.where(mr, t_c, 0), axis=0, keepdims=True).astype(f32)
    in_r = jnp.sum(mr.astype(jnp.int32), axis=0, keepdims=True) > 0

    seg_c = col_scr[:, 2:3]
    d2 = ((col_scr[:, 0:1] - cx_r) ** 2
          + (col_scr[:, 1:2] - cy_r) ** 2)
    same = (seg_c == seg_r) & (seg_c >= 0) & in_r
    a = jnp.where(same, jnp.exp(-d2), 0.0)
    a = a + (idx_c == idx_r).astype(f32)

    deg_b = jnp.dot(jnp.ones((1, n), f32), a,
                    preferred_element_type=f32)
    rsum = jnp.dot(a, jnp.ones((block_n, 1), f32),
                   preferred_element_type=f32)
    contrib = jnp.dot(a * jax.lax.rsqrt(deg_b), ys,
                      preferred_element_type=f32)

    @pl.when(s == 0)
    def _():
        hacc[...] = contrib
        col_scr[:, 3:4] = rsum

    @pl.when(s != 0)
    def _():
        hacc[...] += contrib
        col_scr[:, 3:4] += rsum

    @pl.when(s == nb - 1)
    def _():
        h = jnp.maximum(
            hacc[...] * jax.lax.rsqrt(col_scr[:, 3:4]) + b_ref[...],
            0.0)
        idx_l = jax.lax.broadcasted_iota(jnp.int32, (1, n), 1)
        mf = ((idx_l >= starts_c) & (idx_l < ends_c)).astype(f32)
        acc = jnp.dot(mf, h, preferred_element_type=f32)
        cnt = jnp.sum(mf, axis=1, keepdims=True)
        out_ref[...] = acc / jnp.maximum(cnt, 1.0)


@jax.jit
def kernel(rois, rois_features, temporal_start_end, W, b):
    n, c, hh, ww = rois_features.shape
    hw = hh * ww
    t = temporal_start_end.shape[0]
    block_n = 128
    x = rois_features.transpose(2, 3, 0, 1).reshape(hw, n, c)
    tse = temporal_start_end.astype(jnp.int32)
    nb = n // block_n
    cxr = ((rois[:, 0] + rois[:, 2]) * 0.5).reshape(nb, 1, block_n)
    cyr = ((rois[:, 1] + rois[:, 3]) * 0.5).reshape(nb, 1, block_n)

    return pl.pallas_call(
        functools.partial(_fused_kernel, n=n, t=t, c=c, block_n=block_n),
        grid=(n // block_n,),
        in_specs=[
            pl.BlockSpec((hw, block_n, c), lambda i: (0, i, 0)),
            pl.BlockSpec((c, c), lambda i: (0, 0)),
            pl.BlockSpec((n, 4), lambda i: (0, 0)),
            pl.BlockSpec((1, 1, block_n), lambda i: (i, 0, 0)),
            pl.BlockSpec((1, 1, block_n), lambda i: (i, 0, 0)),
            pl.BlockSpec((t, 2), lambda i: (0, 0)),
            pl.BlockSpec((2, t), lambda i: (0, 0)),
            pl.BlockSpec((1, c), lambda i: (0, 0)),
        ],
        out_specs=pl.BlockSpec((t, c), lambda i: (0, 0)),
        out_shape=jax.ShapeDtypeStruct((t, c), jnp.float32),
        scratch_shapes=[
            pltpu.VMEM((n, c), jnp.float32),
            pltpu.VMEM((n, 4), jnp.float32),
        ],
    )(x, W, rois, cxr, cyr, tse, tse.T, b.reshape(1, c).astype(jnp.float32))
